# speedup vs baseline: 2.4227x; 2.4227x over previous
import jax
import jax.numpy as jnp
from jax.experimental import pallas as pl


def _add_kernel(x_ref, p_ref, o_ref):
    o_ref[...] = x_ref[...] + p_ref[...]


def kernel(inputs, pos_table):
    batch, seq_len, d_model = inputs.shape
    bs = 512
    n_s = seq_len // bs
    grid = (batch, n_s)
    return pl.pallas_call(
        _add_kernel,
        grid=grid,
        in_specs=[
            pl.BlockSpec((1, bs, d_model), lambda b, s: (b, s, 0)),
            pl.BlockSpec((bs, d_model), lambda b, s: (s, 0)),
        ],
        out_specs=pl.BlockSpec((1, bs, d_model), lambda b, s: (b, s, 0)),
        out_shape=jax.ShapeDtypeStruct(inputs.shape, inputs.dtype),
    )(inputs, pos_table)
